# baseline (speedup 1.0000x reference)
import functools

import jax
import jax.numpy as jnp
from jax import lax
from jax.experimental import pallas as pl
from jax.experimental.pallas import tpu as pltpu
from jax.experimental.pallas import tpu_sc as plsc

_B = 4096
_V = 100000
_D = 64
_L = 16
_NC, _NS = 2, 16
_NW = _NC * _NS
_BPW = _B // _NW
_C = 32
_NCHUNK = _BPW // _C
_NSCORE = 21
_UROWS = _NSCORE * _C
_UIW = 96
_UIR = _UROWS // _UIW


def _scv_body(cidx_hbm, v_hbm, out_hbm, cidx_v, vrows, vout, sem):
    wid = lax.axis_index("s") * _NC + lax.axis_index("c")
    base = wid * _BPW
    pltpu.sync_copy(cidx_hbm.at[pl.ds(base * 4, _BPW * 4)], cidx_v)
    cps = []
    for i in range(4):
        sl = pl.ds(i * _BPW, _BPW)
        cps.append(pltpu.async_copy(v_hbm.at[cidx_v.at[sl]], vrows.at[sl],
                                    sem))
    for cp in cps:
        cp.wait()

    def elem(c, carry):
        for k in range(4):
            s = pl.ds(16 * k, 16)
            vout[c, s] = (vrows[4 * c, s] + vrows[4 * c + 1, s]
                          + vrows[4 * c + 2, s] + vrows[4 * c + 3, s]) * 0.25
        return carry

    lax.fori_loop(0, _BPW, elem, 0)
    pltpu.sync_copy(vout, out_hbm.at[pl.ds(base, _BPW)])


_scv_call = functools.partial(
    pl.kernel,
    out_type=jax.ShapeDtypeStruct((_B, _D), jnp.float32),
    mesh=plsc.VectorSubcoreMesh(core_axis_name="c", subcore_axis_name="s"),
    scratch_types=[
        pltpu.VMEM((_BPW * 4,), jnp.int32),
        pltpu.VMEM((_BPW * 4, _D), jnp.float32),
        pltpu.VMEM((_BPW, _D), jnp.float32),
        pltpu.SemaphoreType.DMA,
    ],
    compiler_params=pltpu.CompilerParams(use_tc_tiling_on_sc=False,
                                         needs_layout_passes=False),
)(_scv_body)


def _sc_body(uidx_hbm, vvec_hbm, u_hbm, out_hbm,
             uidx_v, vv, urows, out_v, sem0, sem1):
    sems = (sem0, sem1)
    wid = lax.axis_index("s") * _NC + lax.axis_index("c")

    def issue(g, b):
        base = wid * _BPW + g * _C
        pltpu.sync_copy(vvec_hbm.at[pl.ds(base, _C)], vv.at[b])
        pltpu.sync_copy(uidx_hbm.at[pl.ds(base * _NSCORE, _UROWS)],
                        uidx_v.at[b])
        cps = []
        for i in range(_UIR):
            sl = pl.ds(i * _UIW, _UIW)
            cps.append(pltpu.async_copy(u_hbm.at[uidx_v.at[b, sl]],
                                        urows.at[b, sl], sems[b]))
        return cps

    lanes = lax.iota(jnp.int32, _L)
    cps = issue(0, 0)
    for g in range(_NCHUNK):
        b = g % 2
        nxt = issue(g + 1, 1 - b) if g + 1 < _NCHUNK else []
        for cp in cps:
            cp.wait()
        cps = nxt

        def elem(c, carry, b=b):
            sl = [pl.ds(16 * k, 16) for k in range(4)]
            v = [vv[b, c, s] for s in sl]
            acc0 = jnp.zeros((_L,), jnp.float32)
            acc1 = jnp.zeros((_L,), jnp.float32)
            row = _NSCORE * c
            for j in range(_NSCORE):
                p = urows[b, row + j, sl[0]] * v[0]
                for k in range(1, 4):
                    p = p + urows[b, row + j, sl[k]] * v[k]
                t = jnp.sum(p)
                t = t if j == 0 else -t
                if j < _L:
                    acc0 = jnp.where(lanes == j, t, acc0)
                else:
                    acc1 = jnp.where(lanes == (j - _L), t, acc1)
            out_v[c, pl.ds(0, _L)] = acc0
            out_v[c, pl.ds(_L, _L)] = acc1
            return carry

        lax.fori_loop(0, _C, elem, 0)
        base = wid * _BPW + g * _C
        pltpu.sync_copy(out_v, out_hbm.at[pl.ds(base, _C)])


_sc_call = functools.partial(
    pl.kernel,
    out_type=jax.ShapeDtypeStruct((_B, 2 * _L), jnp.float32),
    mesh=plsc.VectorSubcoreMesh(core_axis_name="c", subcore_axis_name="s"),
    scratch_types=[
        pltpu.VMEM((2, _UROWS), jnp.int32),
        pltpu.VMEM((2, _C, _D), jnp.float32),
        pltpu.VMEM((2, _UROWS, _D), jnp.float32),
        pltpu.VMEM((_C, 2 * _L), jnp.float32),
        pltpu.SemaphoreType.DMA,
        pltpu.SemaphoreType.DMA,
    ],
    compiler_params=pltpu.CompilerParams(use_tc_tiling_on_sc=False,
                                         needs_layout_passes=False),
)(_sc_body)


def _tc_body(x_ref, o_ref):
    x = x_ref[...]
    col = lax.broadcasted_iota(jnp.int32, x.shape, 1)
    ls = jnp.minimum(x, 0.0) - jnp.log1p(jnp.exp(-jnp.abs(x)))
    ls = jnp.where((col & (2 * _L - 1)) < _NSCORE, ls, 0.0)
    o_ref[...] = jnp.full((1, 1), -jnp.sum(ls) / _B, jnp.float32)


_tc_call = pl.pallas_call(
    _tc_body,
    out_shape=jax.ShapeDtypeStruct((1, 1), jnp.float32),
)


def kernel(center_words, target_words, neg_words, V_w, U_w):
    cidx = center_words.astype(jnp.int32).reshape(-1)
    uidx = jnp.concatenate(
        [target_words.astype(jnp.int32), neg_words.astype(jnp.int32)],
        axis=1).reshape(-1)
    vvec = _scv_call(cidx, V_w)
    scores = _sc_call(uidx, vvec, U_w)
    loss = _tc_call(scores.reshape(_B // 4, 8 * _L))
    return loss[0, 0]


# speedup vs baseline: 1.0199x; 1.0199x over previous
import functools

import jax
import jax.numpy as jnp
from jax import lax
from jax.experimental import pallas as pl
from jax.experimental.pallas import tpu as pltpu
from jax.experimental.pallas import tpu_sc as plsc

_B = 4096
_V = 100000
_D = 64
_L = 16
_NC, _NS = 2, 16
_NW = _NC * _NS
_BPW = _B // _NW
_C = 32
_NCHUNK = _BPW // _C
_NSCORE = 21
_UROWS = _NSCORE * _C
_UIW = 96
_UIR = _UROWS // _UIW


def _sc_body(cidx_hbm, uidx_hbm, v_hbm, u_hbm, out_hbm,
             cidx_v, uidx_v, vrows, urows, out_v, sem0, sem1):
    sems = (sem0, sem1)
    wid = lax.axis_index("s") * _NC + lax.axis_index("c")

    def issue(g, b):
        base = wid * _BPW + g * _C
        pltpu.sync_copy(cidx_hbm.at[pl.ds(base * 4, _C * 4)], cidx_v.at[b])
        pltpu.sync_copy(uidx_hbm.at[pl.ds(base * _NSCORE, _UROWS)],
                        uidx_v.at[b])
        cps = [pltpu.async_copy(v_hbm.at[cidx_v.at[b]], vrows.at[b],
                                sems[b])]
        for i in range(_UIR):
            sl = pl.ds(i * _UIW, _UIW)
            cps.append(pltpu.async_copy(u_hbm.at[uidx_v.at[b, sl]],
                                        urows.at[b, sl], sems[b]))
        return cps

    lanes = lax.iota(jnp.int32, _L)
    cps = issue(0, 0)
    for g in range(_NCHUNK):
        b = g % 2
        nxt = issue(g + 1, 1 - b) if g + 1 < _NCHUNK else []
        for cp in cps:
            cp.wait()
        cps = nxt

        def elem(c, carry, b=b):
            sl = [pl.ds(16 * k, 16) for k in range(4)]
            v = [(vrows[b, 4 * c, s] + vrows[b, 4 * c + 1, s]
                  + vrows[b, 4 * c + 2, s] + vrows[b, 4 * c + 3, s]) * 0.25
                 for s in sl]
            acc0 = jnp.zeros((_L,), jnp.float32)
            acc1 = jnp.zeros((_L,), jnp.float32)
            row = _NSCORE * c
            for j in range(_NSCORE):
                p = urows[b, row + j, sl[0]] * v[0]
                for k in range(1, 4):
                    p = p + urows[b, row + j, sl[k]] * v[k]
                t = jnp.sum(p)
                t = t if j == 0 else -t
                if j < _L:
                    acc0 = jnp.where(lanes == j, t, acc0)
                else:
                    acc1 = jnp.where(lanes == (j - _L), t, acc1)
            out_v[c, pl.ds(0, _L)] = acc0
            out_v[c, pl.ds(_L, _L)] = acc1
            return carry

        lax.fori_loop(0, _C, elem, 0)
        base = wid * _BPW + g * _C
        pltpu.sync_copy(out_v, out_hbm.at[pl.ds(base, _C)])


_sc_call = functools.partial(
    pl.kernel,
    out_type=jax.ShapeDtypeStruct((_B, 2 * _L), jnp.float32),
    mesh=plsc.VectorSubcoreMesh(core_axis_name="c", subcore_axis_name="s"),
    scratch_types=[
        pltpu.VMEM((2, _C * 4), jnp.int32),
        pltpu.VMEM((2, _UROWS), jnp.int32),
        pltpu.VMEM((2, _C * 4, _D), jnp.float32),
        pltpu.VMEM((2, _UROWS, _D), jnp.float32),
        pltpu.VMEM((_C, 2 * _L), jnp.float32),
        pltpu.SemaphoreType.DMA,
        pltpu.SemaphoreType.DMA,
    ],
    compiler_params=pltpu.CompilerParams(use_tc_tiling_on_sc=False,
                                         needs_layout_passes=False),
)(_sc_body)


def _tc_body(x_ref, o_ref):
    x = x_ref[...]
    col = lax.broadcasted_iota(jnp.int32, x.shape, 1)
    ls = jnp.minimum(x, 0.0) - jnp.log1p(jnp.exp(-jnp.abs(x)))
    ls = jnp.where((col & (2 * _L - 1)) < _NSCORE, ls, 0.0)
    o_ref[...] = jnp.full((1, 1), -jnp.sum(ls) / _B, jnp.float32)


_tc_call = pl.pallas_call(
    _tc_body,
    out_shape=jax.ShapeDtypeStruct((1, 1), jnp.float32),
)


def kernel(center_words, target_words, neg_words, V_w, U_w):
    cidx = center_words.astype(jnp.int32).reshape(-1)
    uidx = jnp.concatenate(
        [target_words.astype(jnp.int32), neg_words.astype(jnp.int32)],
        axis=1).reshape(-1)
    scores = _sc_call(cidx, uidx, V_w, U_w)
    loss = _tc_call(scores.reshape(_B // 4, 8 * _L))
    return loss[0, 0]
